# baseline (speedup 1.0000x reference)
import jax
import jax.numpy as jnp
from jax.experimental import pallas as pl
from jax.experimental.pallas import tpu as pltpu

EMB = 256
MAXN = 2048
BSIZE = 8
LENGTHS = (512, 768, 1024, 1024, 1024, 1280, 1280, 1280)
TILE = 256
NBLK = tuple(l // TILE for l in LENGTHS)
SEQ_OF_BLK = tuple(s for s in range(BSIZE) for _ in range(NBLK[s]))
JLOC_OF_BLK = tuple(j for s in range(BSIZE) for j in range(NBLK[s]))
NVALID = sum(NBLK)


def _lookup(table, i):
    v = jnp.int32(table[-1])
    for k in range(len(table) - 2, -1, -1):
        v = jnp.where(i == k, jnp.int32(table[k]), v)
    return v


def _body(xraw_ref, xmp_ref, meta_ref, w6_ref, w7_ref, w5pi_ref, w5v_ref,
          b6_ref, b7_ref, b5pi_ref, b5v_ref, actor_ref, critic_ref,
          w6t_s, w7t_s, w5a_s, w5b_s):
    i = pl.program_id(0)
    s = _lookup(SEQ_OF_BLK, i)
    jloc = _lookup(JLOC_OF_BLK, i)
    first = jloc == 0

    @pl.when(i == 0)
    def _prep():
        w6t_s[...] = jnp.transpose(w6_ref[...])
        w7t_s[...] = jnp.transpose(w7_ref[...])
        w5a_s[:, 0:1] = jnp.transpose(w5pi_ref[:, :EMB])
        w5a_s[:, 1:2] = jnp.transpose(w5v_ref[:, :EMB])
        w5b_s[:, 0:1] = jnp.transpose(w5pi_ref[:, EMB:])
        w5b_s[:, 1:2] = jnp.transpose(w5v_ref[:, EMB:])

    @pl.when(first)
    def _fill():
        actor_ref[...] = jnp.full((1, MAXN, 1), -1e20, jnp.float32)

    g = jnp.maximum(
        jnp.dot(xmp_ref[...], w6t_s[...],
                preferred_element_type=jnp.float32) + b6_ref[...], 0.0)
    l = jnp.maximum(
        jnp.dot(xraw_ref[...], w7t_s[...],
                preferred_element_type=jnp.float32) + b7_ref[...], 0.0)
    p = (jnp.dot(g, w5a_s[...], preferred_element_type=jnp.float32)
         + jnp.dot(l, w5b_s[...], preferred_element_type=jnp.float32))
    logits = (p[:, 0:1] + b5pi_ref[0, 0]).reshape(1, TILE, 1)
    actor_ref[:, pl.ds(jloc * TILE, TILE), :] = logits

    q = jnp.where(meta_ref[:, 1] != 0.0, p[:, 1] + b5v_ref[0, 0], -1e20)
    m = jnp.max(q)
    prev = jnp.where(first, -jnp.inf, critic_ref[pl.ds(s, 1), :])
    critic_ref[pl.ds(s, 1), :] = jnp.maximum(prev, m)


def kernel(features, W5pi, b5pi, W6pi, b6pi, W7pi, b7pi, W5v, b5v):
    actor, critic = pl.pallas_call(
        _body,
        grid=(NVALID,),
        in_specs=[
            pl.BlockSpec((TILE, EMB), lambda i: (i, 0)),
            pl.BlockSpec((TILE, EMB), lambda i: (i, 1)),
            pl.BlockSpec((TILE, 128), lambda i: (i, 4)),
            pl.BlockSpec((EMB, EMB), lambda i: (0, 0)),
            pl.BlockSpec((EMB, EMB), lambda i: (0, 0)),
            pl.BlockSpec((1, 2 * EMB), lambda i: (0, 0)),
            pl.BlockSpec((1, 2 * EMB), lambda i: (0, 0)),
            pl.BlockSpec((1, EMB), lambda i: (0, 0)),
            pl.BlockSpec((1, EMB), lambda i: (0, 0)),
            pl.BlockSpec((1, 1), lambda i: (0, 0)),
            pl.BlockSpec((1, 1), lambda i: (0, 0)),
        ],
        out_specs=[
            pl.BlockSpec((1, MAXN, 1), lambda i: (_lookup(SEQ_OF_BLK, i), 0, 0)),
            pl.BlockSpec((BSIZE, 1), lambda i: (0, 0)),
        ],
        out_shape=[
            jax.ShapeDtypeStruct((BSIZE, MAXN, 1), jnp.float32),
            jax.ShapeDtypeStruct((BSIZE, 1), jnp.float32),
        ],
        scratch_shapes=[
            pltpu.VMEM((EMB, EMB), jnp.float32),
            pltpu.VMEM((EMB, EMB), jnp.float32),
            pltpu.VMEM((EMB, 2), jnp.float32),
            pltpu.VMEM((EMB, 2), jnp.float32),
        ],
    )(features, features, features, W6pi, W7pi,
      W5pi, W5v, b6pi.reshape(1, EMB), b7pi.reshape(1, EMB),
      b5pi.reshape(1, 1), b5v.reshape(1, 1))

    return actor, critic


# speedup vs baseline: 1.3141x; 1.3141x over previous
import jax
import jax.numpy as jnp
from jax.experimental import pallas as pl
from jax.experimental.pallas import tpu as pltpu

EMB = 256
MAXN = 2048
BSIZE = 8
LENGTHS = (512, 768, 1024, 1024, 1024, 1280, 1280, 1280)
TOTAL = 8192
CHUNK = 256
NBLK = tuple(l // CHUNK for l in LENGTHS)
SEQ_OF_BLK = tuple(s for s in range(BSIZE) for _ in range(NBLK[s]))
JLOC_OF_BLK = tuple(j for s in range(BSIZE) for j in range(NBLK[s]))
TILE = 1024
NSTEP = TOTAL // TILE
CPS = TILE // CHUNK


def _lookup(table, i):
    v = jnp.int32(table[-1])
    for k in range(len(table) - 2, -1, -1):
        v = jnp.where(i == k, jnp.int32(table[k]), v)
    return v


def _body(xraw_ref, xmp_ref, meta_ref, w6_ref, w7_ref, w5pi_ref, w5v_ref,
          b6_ref, b7_ref, b5pi_ref, b5v_ref, actor_ref, critic_ref,
          llog_s, w6t_s, w7t_s, w5a_s, w5b_s):
    i = pl.program_id(0)

    @pl.when(i == 0)
    def _prep():
        w6t_s[...] = jnp.transpose(w6_ref[...])
        w7t_s[...] = jnp.transpose(w7_ref[...])
        w5a_s[:, 0:1] = jnp.transpose(w5pi_ref[:, :EMB])
        w5a_s[:, 1:2] = jnp.transpose(w5v_ref[:, :EMB])
        w5b_s[:, 0:1] = jnp.transpose(w5pi_ref[:, EMB:])
        w5b_s[:, 1:2] = jnp.transpose(w5v_ref[:, EMB:])

    g = jnp.maximum(
        jnp.dot(xmp_ref[...], w6t_s[...],
                preferred_element_type=jnp.float32) + b6_ref[...], 0.0)
    l = jnp.maximum(
        jnp.dot(xraw_ref[...], w7t_s[...],
                preferred_element_type=jnp.float32) + b7_ref[...], 0.0)
    p = (jnp.dot(g, w5a_s[...], preferred_element_type=jnp.float32)
         + jnp.dot(l, w5b_s[...], preferred_element_type=jnp.float32))
    llog_s[pl.ds(i * TILE, TILE), :] = p[:, 0:1] + b5pi_ref[0, 0]

    q = jnp.where(meta_ref[:, 1] != 0.0, p[:, 1] + b5v_ref[0, 0], -1e20)
    for k in range(CPS):
        c = CPS * i + k
        s = _lookup(SEQ_OF_BLK, c)
        first = _lookup(JLOC_OF_BLK, c) == 0
        m = jnp.max(q[k * CHUNK:(k + 1) * CHUNK])
        prev = jnp.where(first, -jnp.inf, critic_ref[pl.ds(s, 1), :])
        critic_ref[pl.ds(s, 1), :] = jnp.maximum(prev, m)

    @pl.when(i == NSTEP - 1)
    def _writeout():
        actor_ref[...] = jnp.full((BSIZE, MAXN, 1), -1e20, jnp.float32)
        for c in range(len(SEQ_OF_BLK)):
            s = SEQ_OF_BLK[c]
            j = JLOC_OF_BLK[c]
            actor_ref[s, j * CHUNK:(j + 1) * CHUNK, 0:1] = (
                llog_s[c * CHUNK:(c + 1) * CHUNK, :])


def kernel(features, W5pi, b5pi, W6pi, b6pi, W7pi, b7pi, W5v, b5v):
    actor, critic = pl.pallas_call(
        _body,
        grid=(NSTEP,),
        in_specs=[
            pl.BlockSpec((TILE, EMB), lambda i: (i, 0)),
            pl.BlockSpec((TILE, EMB), lambda i: (i, 1)),
            pl.BlockSpec((TILE, 128), lambda i: (i, 4)),
            pl.BlockSpec((EMB, EMB), lambda i: (0, 0)),
            pl.BlockSpec((EMB, EMB), lambda i: (0, 0)),
            pl.BlockSpec((1, 2 * EMB), lambda i: (0, 0)),
            pl.BlockSpec((1, 2 * EMB), lambda i: (0, 0)),
            pl.BlockSpec((1, EMB), lambda i: (0, 0)),
            pl.BlockSpec((1, EMB), lambda i: (0, 0)),
            pl.BlockSpec((1, 1), lambda i: (0, 0)),
            pl.BlockSpec((1, 1), lambda i: (0, 0)),
        ],
        out_specs=[
            pl.BlockSpec((BSIZE, MAXN, 1), lambda i: (0, 0, 0)),
            pl.BlockSpec((BSIZE, 1), lambda i: (0, 0)),
        ],
        out_shape=[
            jax.ShapeDtypeStruct((BSIZE, MAXN, 1), jnp.float32),
            jax.ShapeDtypeStruct((BSIZE, 1), jnp.float32),
        ],
        scratch_shapes=[
            pltpu.VMEM((TOTAL, 1), jnp.float32),
            pltpu.VMEM((EMB, EMB), jnp.float32),
            pltpu.VMEM((EMB, EMB), jnp.float32),
            pltpu.VMEM((EMB, 2), jnp.float32),
            pltpu.VMEM((EMB, 2), jnp.float32),
        ],
    )(features, features, features, W6pi, W7pi,
      W5pi, W5v, b6pi.reshape(1, EMB), b7pi.reshape(1, EMB),
      b5pi.reshape(1, 1), b5v.reshape(1, 1))

    return actor, critic


# speedup vs baseline: 1.5736x; 1.1975x over previous
import numpy as np
import jax
import jax.numpy as jnp
from jax.experimental import pallas as pl
from jax.experimental.pallas import tpu as pltpu

EMB = 256
MAXN = 2048
BSIZE = 8
LENGTHS = (512, 768, 1024, 1024, 1024, 1280, 1280, 1280)
TOTAL = 8192
CHUNK = 256
NBLK = tuple(l // CHUNK for l in LENGTHS)
SEQ_OF_BLK = tuple(s for s in range(BSIZE) for _ in range(NBLK[s]))
JLOC_OF_BLK = tuple(j for s in range(BSIZE) for j in range(NBLK[s]))
OUT_ROW = tuple(8 * s + j for s, j in zip(SEQ_OF_BLK, JLOC_OF_BLK))
TILE = 1024
HTILE = TILE // 2
NSTEP = TOTAL // TILE
CPS = TILE // CHUNK
_PAD_MASK = np.arange(MAXN)[None, :, None] >= np.asarray(LENGTHS)[:, None, None]


def _lookup(table, i):
    v = jnp.int32(table[-1])
    for k in range(len(table) - 2, -1, -1):
        v = jnp.where(i == k, jnp.int32(table[k]), v)
    return v


def _body(xraw0_ref, xraw1_ref, xmp0_ref, xmp1_ref, meta_ref,
          w6_ref, w7_ref, w5pi_ref, w5v_ref,
          b6_ref, b7_ref, b5pi_ref, b5v_ref, actor_ref, critic_ref,
          w6t_s, w7t_s, w5a_s, w5b_s):
    i = pl.program_id(0)

    @pl.when(i == 0)
    def _prep():
        w6t_s[...] = jnp.transpose(w6_ref[...])
        w7t_s[...] = jnp.transpose(w7_ref[...])
        w5a_s[:, 0:1] = jnp.transpose(w5pi_ref[:, :EMB])
        w5a_s[:, 1:2] = jnp.transpose(w5v_ref[:, :EMB])
        w5b_s[:, 0:1] = jnp.transpose(w5pi_ref[:, EMB:])
        w5b_s[:, 1:2] = jnp.transpose(w5v_ref[:, EMB:])
        actor_ref[...] = jnp.full((BSIZE * 8, CHUNK), -1e20, jnp.float32)

    for h, (xraw_ref, xmp_ref) in enumerate(
            ((xraw0_ref, xmp0_ref), (xraw1_ref, xmp1_ref))):
        g = jnp.maximum(
            jnp.dot(xmp_ref[...], w6t_s[...],
                    preferred_element_type=jnp.float32) + b6_ref[...], 0.0)
        l = jnp.maximum(
            jnp.dot(xraw_ref[...], w7t_s[...],
                    preferred_element_type=jnp.float32) + b7_ref[...], 0.0)
        p = (jnp.dot(g, w5a_s[...], preferred_element_type=jnp.float32)
             + jnp.dot(l, w5b_s[...], preferred_element_type=jnp.float32))
        t = jnp.transpose(p[:, 0:1] + b5pi_ref[0, 0])
        q = jnp.where(meta_ref[h * HTILE:(h + 1) * HTILE, 1] != 0.0,
                      p[:, 1] + b5v_ref[0, 0], -1e20)
        for k in range(HTILE // CHUNK):
            c = CPS * i + (HTILE // CHUNK) * h + k
            out_row = _lookup(OUT_ROW, c)
            actor_ref[pl.ds(out_row, 1), :] = t[0:1, k * CHUNK:(k + 1) * CHUNK]
            s = _lookup(SEQ_OF_BLK, c)
            first = _lookup(JLOC_OF_BLK, c) == 0
            m = jnp.max(q[k * CHUNK:(k + 1) * CHUNK])
            prev = jnp.where(first, -jnp.inf, critic_ref[pl.ds(s, 1), :])
            critic_ref[pl.ds(s, 1), :] = jnp.maximum(prev, m)


def kernel(features, W5pi, b5pi, W6pi, b6pi, W7pi, b7pi, W5v, b5v):
    actor64, critic = pl.pallas_call(
        _body,
        grid=(NSTEP,),
        in_specs=[
            pl.BlockSpec((HTILE, EMB), lambda i: (2 * i, 0)),
            pl.BlockSpec((HTILE, EMB), lambda i: (2 * i + 1, 0)),
            pl.BlockSpec((HTILE, EMB), lambda i: (2 * i, 1)),
            pl.BlockSpec((HTILE, EMB), lambda i: (2 * i + 1, 1)),
            pl.BlockSpec((TILE, 128), lambda i: (i, 4)),
            pl.BlockSpec((EMB, EMB), lambda i: (0, 0)),
            pl.BlockSpec((EMB, EMB), lambda i: (0, 0)),
            pl.BlockSpec((1, 2 * EMB), lambda i: (0, 0)),
            pl.BlockSpec((1, 2 * EMB), lambda i: (0, 0)),
            pl.BlockSpec((1, EMB), lambda i: (0, 0)),
            pl.BlockSpec((1, EMB), lambda i: (0, 0)),
            pl.BlockSpec((1, 1), lambda i: (0, 0)),
            pl.BlockSpec((1, 1), lambda i: (0, 0)),
        ],
        out_specs=[
            pl.BlockSpec((BSIZE * 8, CHUNK), lambda i: (0, 0)),
            pl.BlockSpec((BSIZE, 1), lambda i: (0, 0)),
        ],
        out_shape=[
            jax.ShapeDtypeStruct((BSIZE * 8, CHUNK), jnp.float32),
            jax.ShapeDtypeStruct((BSIZE, 1), jnp.float32),
        ],
        scratch_shapes=[
            pltpu.VMEM((EMB, EMB), jnp.float32),
            pltpu.VMEM((EMB, EMB), jnp.float32),
            pltpu.VMEM((EMB, 2), jnp.float32),
            pltpu.VMEM((EMB, 2), jnp.float32),
        ],
    )(features, features, features, features, features, W6pi, W7pi,
      W5pi, W5v, b6pi.reshape(1, EMB), b7pi.reshape(1, EMB),
      b5pi.reshape(1, 1), b5v.reshape(1, 1))

    actor = jnp.where(_PAD_MASK, jnp.float32(-1e20),
                      actor64.reshape(BSIZE, MAXN, 1))
    return actor, critic
